# baseline (speedup 1.0000x reference)
import functools

import jax
import jax.numpy as jnp
from jax import lax
from jax.experimental import pallas as pl
from jax.experimental.pallas import tpu as pltpu
from jax.experimental.pallas import tpu_sc as plsc

N = 10000
NPAD = 10240
F = 128
H = 64
C = 40
E = 320000
NB = 7
NKNOT = 11
KORD = 3

NCORE = 2
NSUB = 16
NWORK = NCORE * NSUB
EK = 128
NCHUNK = 80
EPW = EK * NCHUNK
EPAD = EPW * NWORK
TRASH = 10016
RPT = NPAD // NSUB
NSLOT = 5
PREF = 4
NGRP = NCHUNK // NSLOT

BLK = 512
NBLK = NPAD // BLK


def _sc_deg_body(col_hbm, ones_hbm, zeros_hbm, out_hbm, idx_v, ones_v, deg_sh):
    c = lax.axis_index("c")
    s = lax.axis_index("s")
    wid = s * NCORE + c
    pltpu.sync_copy(zeros_hbm.at[pl.ds(s * RPT, RPT)],
                    deg_sh.at[pl.ds(s * RPT, RPT)])
    pltpu.sync_copy(col_hbm.at[wid], idx_v)
    pltpu.sync_copy(ones_hbm, ones_v)
    plsc.subcore_barrier()

    def body(i, carry):
        pltpu.sync_copy(ones_v, deg_sh.at[idx_v.at[i]], add=True)
        return carry

    lax.fori_loop(0, NCHUNK, body, 0)
    plsc.subcore_barrier()
    pltpu.sync_copy(deg_sh.at[pl.ds(s * RPT, RPT)],
                    out_hbm.at[c, pl.ds(s * RPT, RPT)])


@functools.cache
def _sc_deg_kernel():
    mesh = plsc.VectorSubcoreMesh(core_axis_name="c", subcore_axis_name="s")
    return pl.kernel(
        _sc_deg_body,
        out_type=jax.ShapeDtypeStruct((NCORE, NPAD), jnp.float32),
        mesh=mesh,
        scratch_types=[
            pltpu.VMEM((NCHUNK, EK), jnp.int32),
            pltpu.VMEM((EK,), jnp.float32),
            pltpu.VMEM_SHARED((NPAD,), jnp.float32),
        ],
    )


def _sc_scat_body(row_hbm, col_hbm, hp_hbm, zeros_hbm, out_hbm,
                  idxr, idxc, rows_v, acc_sh, *sems):
    c = lax.axis_index("c")
    s = lax.axis_index("s")
    wid = s * NCORE + c
    pltpu.sync_copy(zeros_hbm.at[pl.ds(s * RPT, RPT)],
                    acc_sh.at[pl.ds(s * RPT, RPT)])
    pltpu.sync_copy(row_hbm.at[wid], idxr)
    pltpu.sync_copy(col_hbm.at[wid], idxc)
    plsc.subcore_barrier()

    gsems = sems[:NSLOT]
    ssems = sems[NSLOT:]

    def _drain_scat(d):
        pltpu.make_async_copy(hp_hbm.at[pl.ds(0, EK)], rows_v.at[d],
                              ssems[d]).wait()

    for j in range(PREF):
        pltpu.async_copy(hp_hbm.at[idxr.at[j]], rows_v.at[j], gsems[j])

    def grp(g, carry):
        for d in range(NSLOT):
            i = g * NSLOT + d
            pltpu.make_async_copy(hp_hbm.at[idxr.at[d]], rows_v.at[d],
                                  gsems[d]).wait()
            pltpu.async_copy(rows_v.at[d], acc_sh.at[idxc.at[i]],
                             ssems[d], add=True)
            d2 = (d + PREF) % NSLOT

            @pl.when(i + PREF < NCHUNK)
            def _():
                @pl.when(i >= NSLOT - PREF)
                def _():
                    _drain_scat(d2)

                pltpu.async_copy(hp_hbm.at[idxr.at[i + PREF]], rows_v.at[d2],
                                 gsems[d2])
        return carry

    lax.fori_loop(0, NGRP, grp, 0)
    for d in range(NSLOT):
        _drain_scat(d)
    plsc.subcore_barrier()
    pltpu.sync_copy(acc_sh.at[pl.ds(s * RPT, RPT)],
                    out_hbm.at[c, pl.ds(s * RPT, RPT)])


@functools.cache
def _sc_scat_kernel():
    mesh = plsc.VectorSubcoreMesh(core_axis_name="c", subcore_axis_name="s")
    return pl.kernel(
        _sc_scat_body,
        out_type=jax.ShapeDtypeStruct((NCORE, NPAD, H), jnp.float32),
        mesh=mesh,
        scratch_types=[
            pltpu.VMEM((NCHUNK, EK), jnp.int32),
            pltpu.VMEM((NCHUNK, EK), jnp.int32),
            pltpu.VMEM((NSLOT, EK, H), jnp.float32),
            pltpu.VMEM_SHARED((NPAD, H), jnp.float32),
        ] + [pltpu.SemaphoreType.DMA] * (2 * NSLOT),
        compiler_params=pltpu.CompilerParams(use_tc_tiling_on_sc=False),
    )


def _bspline_bases(xb, g):
    b = [jnp.where((xb >= g[j]) & (xb < g[j + 1]), 1.0, 0.0)
         for j in range(NKNOT - 1)]
    for p in range(1, KORD + 1):
        nxt = []
        for j in range(NKNOT - 1 - p):
            left = (xb - g[j]) / (g[j + p] - g[j]) * b[j]
            right = (g[j + p + 1] - xb) / (g[j + p + 1] - g[j + 1]) * b[j + 1]
            nxt.append(left + right)
        b = nxt
    return b


def _dot(a, b):
    return lax.dot_general(a, b, (((1,), (0,)), ((), ())),
                           precision=lax.Precision.HIGHEST,
                           preferred_element_type=jnp.float32)


def _kan_block(xb, g_ref, bw_ref, sw_ref, sc_ref):
    g = [g_ref[j] for j in range(NKNOT)]
    silu = xb * (1.0 / (1.0 + jnp.exp(-xb)))
    out = _dot(silu, bw_ref[...])
    bases = _bspline_bases(xb, g)
    scw = sc_ref[...]
    for j in range(NB):
        out = out + _dot(bases[j], sw_ref[j] * scw)
    return out


def _kan1_body(x_ref, d0_ref, d1_ref, g_ref, bw_ref, sw_ref, sc_ref,
               hp_ref, dinv_ref):
    i = pl.program_id(0)
    deg = d0_ref[...] + d1_ref[...] + 1.0
    dinv = lax.rsqrt(deg)
    h = _kan_block(x_ref[...], g_ref, bw_ref, sw_ref, sc_ref)
    rid = i * BLK + lax.broadcasted_iota(jnp.int32, (BLK, 1), 0)
    mask = jnp.where(rid < N, 1.0, 0.0)
    hp_ref[...] = h * dinv * mask
    dinv_ref[...] = dinv


def _kan2_body(x_ref, dinv_ref, g_ref, bw_ref, sw_ref, sc_ref, hp_ref):
    i = pl.program_id(0)
    h = _kan_block(x_ref[...], g_ref, bw_ref, sw_ref, sc_ref)
    rid = i * BLK + lax.broadcasted_iota(jnp.int32, (BLK, 1), 0)
    mask = jnp.where(rid < N, 1.0, 0.0)
    hp_ref[...] = h * dinv_ref[...] * mask


def _bn_body(a0_ref, a1_ref, hp_ref, dinv_ref, bias_ref, gamma_ref, beta_ref,
             out_ref):
    o = (a0_ref[...] + a1_ref[...] + hp_ref[...]) * dinv_ref[...] + bias_ref[...]
    rid = lax.broadcasted_iota(jnp.int32, (NPAD, 1), 0)
    mask = jnp.where(rid < N, 1.0, 0.0)
    mean = jnp.sum(o * mask, axis=0, keepdims=True) * (1.0 / N)
    var = jnp.sum(((o - mean) ** 2) * mask, axis=0, keepdims=True) * (1.0 / N)
    out_ref[...] = (o - mean) * lax.rsqrt(var + 1e-5) * gamma_ref[...] + beta_ref[...]


def _kanout_body(x_ref, b1_ref, b2_ref, g_ref,
                 bw0_ref, sw0_ref, sc0_ref,
                 bw1_ref, sw1_ref, sc1_ref,
                 bw2_ref, sw2_ref, sc2_ref, out_ref):
    o = _kan_block(x_ref[...], g_ref, bw0_ref, sw0_ref, sc0_ref)
    o = o + _kan_block(b1_ref[...], g_ref, bw1_ref, sw1_ref, sc1_ref)
    o = o + _kan_block(b2_ref[...], g_ref, bw2_ref, sw2_ref, sc2_ref)
    out_ref[...] = o


def _row_spec(width):
    return pl.BlockSpec((BLK, width), lambda i: (i, 0))


def _full_spec(shape):
    nd = len(shape)
    return pl.BlockSpec(shape, lambda i: (0,) * nd)


def _smem_spec():
    return pl.BlockSpec(memory_space=pltpu.SMEM)


def _kan1_call(x, d0, d1, grow, bw, sw, sc):
    return pl.pallas_call(
        _kan1_body,
        grid=(NBLK,),
        in_specs=[_row_spec(F), _row_spec(1), _row_spec(1), _smem_spec(),
                  _full_spec((F, H)), _full_spec((NB, F, H)), _full_spec((F, H))],
        out_specs=[_row_spec(H), _row_spec(1)],
        out_shape=[jax.ShapeDtypeStruct((NPAD, H), jnp.float32),
                   jax.ShapeDtypeStruct((NPAD, 1), jnp.float32)],
    )(x, d0, d1, grow, bw, sw, sc)


def _kan2_call(x, dinv, grow, bw, sw, sc):
    return pl.pallas_call(
        _kan2_body,
        grid=(NBLK,),
        in_specs=[_row_spec(H), _row_spec(1), _smem_spec(),
                  _full_spec((H, H)), _full_spec((NB, H, H)), _full_spec((H, H))],
        out_specs=_row_spec(H),
        out_shape=jax.ShapeDtypeStruct((NPAD, H), jnp.float32),
    )(x, dinv, grow, bw, sw, sc)


def _bn_call(a0, a1, hp, dinv, bias, gamma, beta):
    fs = lambda shape: pl.BlockSpec(shape, lambda: (0,) * len(shape))
    return pl.pallas_call(
        _bn_body,
        in_specs=[fs((NPAD, H)), fs((NPAD, H)), fs((NPAD, H)), fs((NPAD, 1)),
                  fs((1, H)), fs((1, H)), fs((1, H))],
        out_specs=fs((NPAD, H)),
        out_shape=jax.ShapeDtypeStruct((NPAD, H), jnp.float32),
    )(a0, a1, hp, dinv, bias, gamma, beta)


def _kanout_call(x, b1, b2, grow, parts):
    in_specs = [_row_spec(F), _row_spec(H), _row_spec(H), _smem_spec()]
    args = [x, b1, b2, grow]
    for (bw, sw, sc) in parts:
        in_specs += [_full_spec(bw.shape), _full_spec(sw.shape),
                     _full_spec(sc.shape)]
        args += [bw, sw, sc]
    return pl.pallas_call(
        _kanout_body,
        grid=(NBLK,),
        in_specs=in_specs,
        out_specs=_row_spec(C),
        out_shape=jax.ShapeDtypeStruct((NPAD, C), jnp.float32),
    )(*args)


def kernel(x, edge_index, grid1, base_w1, spline_w1, scaler1, bias1, gamma1,
           beta1, grid2, base_w2, spline_w2, scaler2, bias2, gamma2, beta2,
           grid_out, base_w_out, spline_w_out, scaler_out):
    f32 = jnp.float32
    x_pad = jnp.zeros((NPAD, F), f32).at[:N].set(x)

    pad = jnp.full((EPAD - E,), TRASH, jnp.int32)
    row = jnp.concatenate([edge_index[0].astype(jnp.int32), pad])
    row = row.reshape(NWORK, NCHUNK, EK)
    col = jnp.concatenate([edge_index[1].astype(jnp.int32), pad])
    col = col.reshape(NWORK, NCHUNK, EK)

    zeros_n = jnp.zeros((NPAD,), f32)
    zeros_nh = jnp.zeros((NPAD, H), f32)
    ones_ek = jnp.ones((EK,), f32)

    bw1 = base_w1.T
    sw1 = jnp.transpose(spline_w1, (2, 1, 0))
    sc1 = scaler1.T
    bw2 = base_w2.T
    sw2 = jnp.transpose(spline_w2, (2, 1, 0))
    sc2 = scaler2.T
    bwo = base_w_out.T
    swo = jnp.transpose(spline_w_out, (2, 1, 0))
    sco = scaler_out.T
    parts = [(bwo[:F], swo[:, :F], sco[:F]),
             (bwo[F:F + H], swo[:, F:F + H], sco[F:F + H]),
             (bwo[F + H:], swo[:, F + H:], sco[F + H:])]

    g1 = grid1[0]
    g2 = grid2[0]
    go = grid_out[0]

    deg = _sc_deg_kernel()(col, ones_ek, zeros_n)
    d0 = deg[0].reshape(NPAD, 1)
    d1 = deg[1].reshape(NPAD, 1)

    hp1, dinv = _kan1_call(x_pad, d0, d1, g1, bw1, sw1, sc1)
    acc1 = _sc_scat_kernel()(row, col, hp1, zeros_nh)
    bn1 = _bn_call(acc1[0], acc1[1], hp1, dinv,
                   bias1.reshape(1, H), gamma1.reshape(1, H), beta1.reshape(1, H))

    hp2 = _kan2_call(bn1, dinv, g2, bw2, sw2, sc2)
    acc2 = _sc_scat_kernel()(row, col, hp2, zeros_nh)
    bn2 = _bn_call(acc2[0], acc2[1], hp2, dinv,
                   bias2.reshape(1, H), gamma2.reshape(1, H), beta2.reshape(1, H))

    out = _kanout_call(x_pad, bn1, bn2, go, parts)
    return out[:N]


# speedup vs baseline: 1.2070x; 1.2070x over previous
import functools

import jax
import jax.numpy as jnp
from jax import lax
from jax.experimental import pallas as pl
from jax.experimental.pallas import tpu as pltpu
from jax.experimental.pallas import tpu_sc as plsc

N = 10000
NPAD = 10240
F = 128
H = 64
C = 40
E = 320000
NB = 7
NKNOT = 11
KORD = 3

NCORE = 2
NSUB = 16
NWORK = NCORE * NSUB
EK = 128
NCHUNK = 80
EPW = EK * NCHUNK
EPAD = EPW * NWORK
TRASH = 10016
RPT = NPAD // NSUB
NSLOT = 5
PREF = 4
NGRP = NCHUNK // NSLOT

BLK = 512
NBLK = NPAD // BLK


def _sc_deg_body(col_hbm, ones_hbm, zeros_hbm, out_hbm, idx_v, ones_v, deg_sh):
    c = lax.axis_index("c")
    s = lax.axis_index("s")
    wid = s * NCORE + c
    pltpu.sync_copy(zeros_hbm.at[pl.ds(s * RPT, RPT)],
                    deg_sh.at[pl.ds(s * RPT, RPT)])
    pltpu.sync_copy(col_hbm.at[wid], idx_v)
    pltpu.sync_copy(ones_hbm, ones_v)
    plsc.subcore_barrier()

    def body(i, carry):
        pltpu.sync_copy(ones_v, deg_sh.at[idx_v.at[i]], add=True)
        return carry

    lax.fori_loop(0, NCHUNK, body, 0)
    plsc.subcore_barrier()
    pltpu.sync_copy(deg_sh.at[pl.ds(s * RPT, RPT)],
                    out_hbm.at[c, pl.ds(s * RPT, RPT)])


@functools.cache
def _sc_deg_kernel():
    mesh = plsc.VectorSubcoreMesh(core_axis_name="c", subcore_axis_name="s")
    return pl.kernel(
        _sc_deg_body,
        out_type=jax.ShapeDtypeStruct((NCORE, NPAD), jnp.float32),
        mesh=mesh,
        scratch_types=[
            pltpu.VMEM((NCHUNK, EK), jnp.int32),
            pltpu.VMEM((EK,), jnp.float32),
            pltpu.VMEM_SHARED((NPAD,), jnp.float32),
        ],
    )


def _sc_scat_body(row_hbm, col_hbm, hp_hbm, zeros_hbm, out_hbm,
                  idxr, idxc, rows_v, acc_sh, *sems):
    c = lax.axis_index("c")
    s = lax.axis_index("s")
    wid = s * NCORE + c
    pltpu.sync_copy(zeros_hbm.at[pl.ds(s * RPT, RPT)],
                    acc_sh.at[pl.ds(s * RPT, RPT)])
    pltpu.sync_copy(row_hbm.at[wid], idxr)
    pltpu.sync_copy(col_hbm.at[wid], idxc)
    plsc.subcore_barrier()

    gsems = sems[:NSLOT]
    ssems = sems[NSLOT:]

    def _drain_scat(d):
        pltpu.make_async_copy(hp_hbm.at[pl.ds(0, EK)], rows_v.at[d],
                              ssems[d]).wait()

    for j in range(PREF):
        pltpu.async_copy(hp_hbm.at[idxr.at[j]], rows_v.at[j], gsems[j])

    def grp(g, carry):
        for d in range(NSLOT):
            i = g * NSLOT + d
            pltpu.make_async_copy(hp_hbm.at[idxr.at[d]], rows_v.at[d],
                                  gsems[d]).wait()
            pltpu.async_copy(rows_v.at[d], acc_sh.at[idxc.at[i]],
                             ssems[d], add=True)
            d2 = (d + PREF) % NSLOT

            @pl.when(i + PREF < NCHUNK)
            def _():
                @pl.when(i >= NSLOT - PREF)
                def _():
                    _drain_scat(d2)

                pltpu.async_copy(hp_hbm.at[idxr.at[i + PREF]], rows_v.at[d2],
                                 gsems[d2])
        return carry

    lax.fori_loop(0, NGRP, grp, 0)
    for d in range(NSLOT):
        _drain_scat(d)
    plsc.subcore_barrier()
    pltpu.sync_copy(acc_sh.at[pl.ds(s * RPT, RPT)],
                    out_hbm.at[c, pl.ds(s * RPT, RPT)])


@functools.cache
def _sc_scat_kernel():
    mesh = plsc.VectorSubcoreMesh(core_axis_name="c", subcore_axis_name="s")
    return pl.kernel(
        _sc_scat_body,
        out_type=jax.ShapeDtypeStruct((NCORE, NPAD, H), jnp.float32),
        mesh=mesh,
        scratch_types=[
            pltpu.VMEM((NCHUNK, EK), jnp.int32),
            pltpu.VMEM((NCHUNK, EK), jnp.int32),
            pltpu.VMEM((NSLOT, EK, H), jnp.float32),
            pltpu.VMEM_SHARED((NPAD, H), jnp.float32),
        ] + [pltpu.SemaphoreType.DMA] * (2 * NSLOT),
        compiler_params=pltpu.CompilerParams(use_tc_tiling_on_sc=False),
    )


def _bspline_bases(xb, g):
    b = [jnp.where((xb >= g[j]) & (xb < g[j + 1]), 1.0, 0.0)
         for j in range(NKNOT - 1)]
    for p in range(1, KORD + 1):
        nxt = []
        for j in range(NKNOT - 1 - p):
            left = (xb - g[j]) / (g[j + p] - g[j]) * b[j]
            right = (g[j + p + 1] - xb) / (g[j + p + 1] - g[j + 1]) * b[j + 1]
            nxt.append(left + right)
        b = nxt
    return b


def _dot(a, b):
    return lax.dot_general(a, b, (((1,), (0,)), ((), ())),
                           preferred_element_type=jnp.float32)


def _kan_block(xb, g_ref, bw_ref, sw_ref, sc_ref):
    g = [g_ref[j] for j in range(NKNOT)]
    silu = xb * (1.0 / (1.0 + jnp.exp(-xb)))
    out = _dot(silu, bw_ref[...])
    bases = _bspline_bases(xb, g)
    scw = sc_ref[...]
    for j in range(NB):
        out = out + _dot(bases[j], sw_ref[j] * scw)
    return out


def _kan1_body(x_ref, d0_ref, d1_ref, g_ref, bw_ref, sw_ref, sc_ref,
               hp_ref, dinv_ref):
    i = pl.program_id(0)
    deg = d0_ref[...] + d1_ref[...] + 1.0
    dinv = lax.rsqrt(deg)
    h = _kan_block(x_ref[...], g_ref, bw_ref, sw_ref, sc_ref)
    rid = i * BLK + lax.broadcasted_iota(jnp.int32, (BLK, 1), 0)
    mask = jnp.where(rid < N, 1.0, 0.0)
    hp_ref[...] = h * dinv * mask
    dinv_ref[...] = dinv


def _kan2_body(x_ref, dinv_ref, g_ref, bw_ref, sw_ref, sc_ref, hp_ref):
    i = pl.program_id(0)
    h = _kan_block(x_ref[...], g_ref, bw_ref, sw_ref, sc_ref)
    rid = i * BLK + lax.broadcasted_iota(jnp.int32, (BLK, 1), 0)
    mask = jnp.where(rid < N, 1.0, 0.0)
    hp_ref[...] = h * dinv_ref[...] * mask


def _bn_body(a0_ref, a1_ref, hp_ref, dinv_ref, bias_ref, gamma_ref, beta_ref,
             out_ref):
    o = (a0_ref[...] + a1_ref[...] + hp_ref[...]) * dinv_ref[...] + bias_ref[...]
    rid = lax.broadcasted_iota(jnp.int32, (NPAD, 1), 0)
    mask = jnp.where(rid < N, 1.0, 0.0)
    mean = jnp.sum(o * mask, axis=0, keepdims=True) * (1.0 / N)
    var = jnp.sum(((o - mean) ** 2) * mask, axis=0, keepdims=True) * (1.0 / N)
    out_ref[...] = (o - mean) * lax.rsqrt(var + 1e-5) * gamma_ref[...] + beta_ref[...]


def _kanout_body(x_ref, b1_ref, b2_ref, g_ref,
                 bw0_ref, sw0_ref, sc0_ref,
                 bw1_ref, sw1_ref, sc1_ref,
                 bw2_ref, sw2_ref, sc2_ref, out_ref):
    o = _kan_block(x_ref[...], g_ref, bw0_ref, sw0_ref, sc0_ref)
    o = o + _kan_block(b1_ref[...], g_ref, bw1_ref, sw1_ref, sc1_ref)
    o = o + _kan_block(b2_ref[...], g_ref, bw2_ref, sw2_ref, sc2_ref)
    out_ref[...] = o


def _row_spec(width):
    return pl.BlockSpec((BLK, width), lambda i: (i, 0))


def _full_spec(shape):
    nd = len(shape)
    return pl.BlockSpec(shape, lambda i: (0,) * nd)


def _smem_spec():
    return pl.BlockSpec(memory_space=pltpu.SMEM)


def _kan1_call(x, d0, d1, grow, bw, sw, sc):
    return pl.pallas_call(
        _kan1_body,
        grid=(NBLK,),
        in_specs=[_row_spec(F), _row_spec(1), _row_spec(1), _smem_spec(),
                  _full_spec((F, H)), _full_spec((NB, F, H)), _full_spec((F, H))],
        out_specs=[_row_spec(H), _row_spec(1)],
        out_shape=[jax.ShapeDtypeStruct((NPAD, H), jnp.float32),
                   jax.ShapeDtypeStruct((NPAD, 1), jnp.float32)],
    )(x, d0, d1, grow, bw, sw, sc)


def _kan2_call(x, dinv, grow, bw, sw, sc):
    return pl.pallas_call(
        _kan2_body,
        grid=(NBLK,),
        in_specs=[_row_spec(H), _row_spec(1), _smem_spec(),
                  _full_spec((H, H)), _full_spec((NB, H, H)), _full_spec((H, H))],
        out_specs=_row_spec(H),
        out_shape=jax.ShapeDtypeStruct((NPAD, H), jnp.float32),
    )(x, dinv, grow, bw, sw, sc)


def _bn_call(a0, a1, hp, dinv, bias, gamma, beta):
    fs = lambda shape: pl.BlockSpec(shape, lambda: (0,) * len(shape))
    return pl.pallas_call(
        _bn_body,
        in_specs=[fs((NPAD, H)), fs((NPAD, H)), fs((NPAD, H)), fs((NPAD, 1)),
                  fs((1, H)), fs((1, H)), fs((1, H))],
        out_specs=fs((NPAD, H)),
        out_shape=jax.ShapeDtypeStruct((NPAD, H), jnp.float32),
    )(a0, a1, hp, dinv, bias, gamma, beta)


def _kanout_call(x, b1, b2, grow, parts):
    in_specs = [_row_spec(F), _row_spec(H), _row_spec(H), _smem_spec()]
    args = [x, b1, b2, grow]
    for (bw, sw, sc) in parts:
        in_specs += [_full_spec(bw.shape), _full_spec(sw.shape),
                     _full_spec(sc.shape)]
        args += [bw, sw, sc]
    return pl.pallas_call(
        _kanout_body,
        grid=(NBLK,),
        in_specs=in_specs,
        out_specs=_row_spec(C),
        out_shape=jax.ShapeDtypeStruct((NPAD, C), jnp.float32),
    )(*args)


def kernel(x, edge_index, grid1, base_w1, spline_w1, scaler1, bias1, gamma1,
           beta1, grid2, base_w2, spline_w2, scaler2, bias2, gamma2, beta2,
           grid_out, base_w_out, spline_w_out, scaler_out):
    f32 = jnp.float32
    x_pad = jnp.zeros((NPAD, F), f32).at[:N].set(x)

    pad = jnp.full((EPAD - E,), TRASH, jnp.int32)
    row = jnp.concatenate([edge_index[0].astype(jnp.int32), pad])
    row = row.reshape(NWORK, NCHUNK, EK)
    col = jnp.concatenate([edge_index[1].astype(jnp.int32), pad])
    col = col.reshape(NWORK, NCHUNK, EK)

    zeros_n = jnp.zeros((NPAD,), f32)
    zeros_nh = jnp.zeros((NPAD, H), f32)
    ones_ek = jnp.ones((EK,), f32)

    bw1 = base_w1.T
    sw1 = jnp.transpose(spline_w1, (2, 1, 0))
    sc1 = scaler1.T
    bw2 = base_w2.T
    sw2 = jnp.transpose(spline_w2, (2, 1, 0))
    sc2 = scaler2.T
    bwo = base_w_out.T
    swo = jnp.transpose(spline_w_out, (2, 1, 0))
    sco = scaler_out.T
    parts = [(bwo[:F], swo[:, :F], sco[:F]),
             (bwo[F:F + H], swo[:, F:F + H], sco[F:F + H]),
             (bwo[F + H:], swo[:, F + H:], sco[F + H:])]

    g1 = grid1[0]
    g2 = grid2[0]
    go = grid_out[0]

    deg = _sc_deg_kernel()(col, ones_ek, zeros_n)
    d0 = deg[0].reshape(NPAD, 1)
    d1 = deg[1].reshape(NPAD, 1)

    hp1, dinv = _kan1_call(x_pad, d0, d1, g1, bw1, sw1, sc1)
    acc1 = _sc_scat_kernel()(row, col, hp1, zeros_nh)
    bn1 = _bn_call(acc1[0], acc1[1], hp1, dinv,
                   bias1.reshape(1, H), gamma1.reshape(1, H), beta1.reshape(1, H))

    hp2 = _kan2_call(bn1, dinv, g2, bw2, sw2, sc2)
    acc2 = _sc_scat_kernel()(row, col, hp2, zeros_nh)
    bn2 = _bn_call(acc2[0], acc2[1], hp2, dinv,
                   bias2.reshape(1, H), gamma2.reshape(1, H), beta2.reshape(1, H))

    out = _kanout_call(x_pad, bn1, bn2, go, parts)
    return out[:N]
